# baseline (speedup 1.0000x reference)
import jax
import jax.numpy as jnp
from jax.experimental import pallas as pl
from jax.experimental.pallas import tpu as pltpu

L, B, D, H, A = 512, 1024, 64, 64, 32
FILL = 0.0
LOG_REWARD_CLIP_MIN = -100.0

L_BLK = 16
N_BLKS = L // L_BLK


def _fused_kernel(states_ref, actions_ref, lengths_ref, logr_ref,
                  w1_ref, b1_ref, w2_ref, b2_ref,
                  pf_out, pb_out, scores_out,
                  accg_ref, tf_ref, tb_ref):
    i = pl.program_id(0)
    lengths = lengths_ref[...]
    w1 = w1_ref[...]
    w2 = w2_ref[...]
    b1 = b1_ref[...]
    b2 = b2_ref[...]

    r8 = jax.lax.broadcasted_iota(jnp.int32, (8, 2 * A), 0)
    c8 = jax.lax.broadcasted_iota(jnp.int32, (8, 2 * A), 1)
    red = (((r8 == 0) & (c8 < A)) | ((r8 == 1) & (c8 >= A))
           ).astype(jnp.float32)

    row = jax.lax.broadcasted_iota(jnp.int32, (2 * A, B), 0)

    acc_g = jnp.zeros((2 * A, B), jnp.float32)
    acc_tf = jnp.zeros((1, B), jnp.float32)
    acc_tb = jnp.zeros((1, B), jnp.float32)

    for j in range(L_BLK):
        x = states_ref[j].astype(jnp.bfloat16)
        h = jnp.maximum(
            jnp.dot(x, w1, preferred_element_type=jnp.float32) + b1, 0.0)
        logits = (jnp.dot(h.astype(jnp.bfloat16), w2,
                          preferred_element_type=jnp.float32)
                  + b2)
        lt = logits.T

        e = jnp.exp(lt)
        s8 = jnp.dot(red, e, preferred_element_type=jnp.float32,
                     precision=jax.lax.Precision.HIGHEST)
        lse = jnp.log(s8[0:2, :])

        a = actions_ref[j]
        step = i * L_BLK + j
        valid = step < lengths
        validb = valid & (step != lengths - 1)
        t_f = jnp.where(valid, a, -1)
        t_b = jnp.where(validb, a + A, -1)
        cond = (row == t_f) | (row == t_b)

        acc_g = acc_g + jnp.where(cond, lt, FILL)
        acc_tf = acc_tf + jnp.where(valid, lse[0:1, :], FILL)
        acc_tb = acc_tb + jnp.where(validb, lse[1:2, :], FILL)

    @pl.when(i == 0)
    def _init():
        accg_ref[...] = acc_g
        tf_ref[...] = acc_tf
        tb_ref[...] = acc_tb

    @pl.when(i > 0)
    def _acc():
        accg_ref[...] += acc_g
        tf_ref[...] += acc_tf
        tb_ref[...] += acc_tb

    @pl.when(i == N_BLKS - 1)
    def _final():
        g = accg_ref[...]
        pf = (jnp.sum(g[:A, :], axis=0, keepdims=True)
              - tf_ref[...])
        pb = (jnp.sum(g[A:, :], axis=0, keepdims=True)
              - tb_ref[...])
        log_r = jnp.maximum(logr_ref[...], LOG_REWARD_CLIP_MIN)
        pf_out[...] = pf
        pb_out[...] = pb
        scores_out[...] = pf - pb - log_r


@jax.jit
def kernel(states, log_rewards, pf_W1, pf_b1, pf_W2, pf_b2,
           pb_W1, pb_b1, pb_W2, pb_b2, actions, lengths):
    w1 = jnp.concatenate([pf_W1, pb_W1], axis=1).astype(jnp.bfloat16)
    b1 = jnp.concatenate([pf_b1, pb_b1])[None, :]
    w2 = jnp.zeros((2 * H, 2 * A), jnp.float32)
    w2 = w2.at[:H, :A].set(pf_W2).at[H:, A:].set(pb_W2)
    w2 = w2.astype(jnp.bfloat16)
    b2 = jnp.concatenate([pf_b2, pb_b2])[None, :]
    actions3 = actions[:, None, :]
    lengths2 = lengths[None, :]
    logr2 = log_rewards[None, :]

    out_shape = [jax.ShapeDtypeStruct((1, B), jnp.float32)] * 3
    rep = pl.BlockSpec((1, B), lambda i: (0, 0))
    pf, pb, scores = pl.pallas_call(
        _fused_kernel,
        grid=(N_BLKS,),
        in_specs=[
            pl.BlockSpec((L_BLK, B, D), lambda i: (i, 0, 0)),
            pl.BlockSpec((L_BLK, 1, B), lambda i: (i, 0, 0)),
            rep,
            rep,
            pl.BlockSpec((D, 2 * H), lambda i: (0, 0)),
            pl.BlockSpec((1, 2 * H), lambda i: (0, 0)),
            pl.BlockSpec((2 * H, 2 * A), lambda i: (0, 0)),
            pl.BlockSpec((1, 2 * A), lambda i: (0, 0)),
        ],
        out_specs=[rep, rep, rep],
        out_shape=out_shape,
        scratch_shapes=[
            pltpu.VMEM((2 * A, B), jnp.float32),
            pltpu.VMEM((1, B), jnp.float32),
            pltpu.VMEM((1, B), jnp.float32),
        ],
        compiler_params=pltpu.CompilerParams(
            dimension_semantics=("arbitrary",),
        ),
    )(states, actions3, lengths2, logr2, w1, b1, w2, b2)
    return pf[0], pb[0], scores[0]


# speedup vs baseline: 2.8611x; 2.8611x over previous
import jax
import jax.numpy as jnp
from jax.experimental import pallas as pl
from jax.experimental.pallas import tpu as pltpu

L, B, D, H, A = 512, 1024, 64, 64, 32
FILL = 0.0
LOG_REWARD_CLIP_MIN = -100.0

L_BLK = 16
N_BLKS = L // L_BLK


def _fused_kernel(states_ref, actions_ref, lengths_ref, logr_ref,
                  w1_ref, b1_ref, w2_ref, b2_ref,
                  pf_out, pb_out, scores_out,
                  gf_ref, gb_ref, tf_ref, tb_ref):
    i = pl.program_id(0)
    lengths = lengths_ref[...]
    w1 = w1_ref[...]
    w2 = w2_ref[...]
    b1 = b1_ref[...]
    b2 = b2_ref[...]

    row = jax.lax.broadcasted_iota(jnp.int32, (2 * A, B), 0)

    acc_gf = jnp.zeros((1, B), jnp.float32)
    acc_gb = jnp.zeros((1, B), jnp.float32)
    acc_tf = jnp.zeros((1, B), jnp.float32)
    acc_tb = jnp.zeros((1, B), jnp.float32)

    for j in range(L_BLK):
        xt = states_ref[j]
        ht = jnp.maximum(
            jnp.dot(w1, xt, preferred_element_type=jnp.float32) + b1,
            0.0).astype(jnp.bfloat16)
        lt = (jnp.dot(w2, ht, preferred_element_type=jnp.float32)
              + b2)

        e = jnp.exp(lt)
        s_f = jnp.sum(e[:A, :], axis=0, keepdims=True)
        s_b = jnp.sum(e[A:, :], axis=0, keepdims=True)
        lse_f = jnp.log(s_f)
        lse_b = jnp.log(s_b)

        a = actions_ref[j]
        step = i * L_BLK + j
        valid = step < lengths
        validb = valid & (step != lengths - 1)
        t_f = jnp.where(valid, a, -1)
        t_b = jnp.where(validb, a + A, -1)
        cond = (row == t_f) | (row == t_b)

        masked = jnp.where(cond, lt, FILL)
        acc_gf = acc_gf + jnp.sum(masked[:A, :], axis=0, keepdims=True)
        acc_gb = acc_gb + jnp.sum(masked[A:, :], axis=0, keepdims=True)
        acc_tf = acc_tf + jnp.where(valid, lse_f, FILL)
        acc_tb = acc_tb + jnp.where(validb, lse_b, FILL)

    @pl.when(i == 0)
    def _init():
        gf_ref[...] = acc_gf
        gb_ref[...] = acc_gb
        tf_ref[...] = acc_tf
        tb_ref[...] = acc_tb

    @pl.when(i > 0)
    def _acc():
        gf_ref[...] += acc_gf
        gb_ref[...] += acc_gb
        tf_ref[...] += acc_tf
        tb_ref[...] += acc_tb

    @pl.when(i == N_BLKS - 1)
    def _final():
        pf = gf_ref[...] - tf_ref[...]
        pb = gb_ref[...] - tb_ref[...]
        log_r = jnp.maximum(logr_ref[...], LOG_REWARD_CLIP_MIN)
        pf_out[...] = pf
        pb_out[...] = pb
        scores_out[...] = pf - pb - log_r


@jax.jit
def kernel(states, log_rewards, pf_W1, pf_b1, pf_W2, pf_b2,
           pb_W1, pb_b1, pb_W2, pb_b2, actions, lengths):
    states_t = jnp.transpose(states, (0, 2, 1))
    w1 = jnp.concatenate([pf_W1, pb_W1], axis=1).T
    b1 = jnp.broadcast_to(
        jnp.concatenate([pf_b1, pb_b1])[:, None], (2 * H, B))
    w2 = jnp.zeros((2 * H, 2 * A), jnp.float32)
    w2 = (w2.at[:H, :A].set(pf_W2).at[H:, A:].set(pb_W2).T
          .astype(jnp.bfloat16))
    b2 = jnp.broadcast_to(
        jnp.concatenate([pf_b2, pb_b2])[:, None], (2 * A, B))
    actions3 = actions[:, None, :]
    lengths2 = lengths[None, :]
    logr2 = log_rewards[None, :]

    out_shape = [jax.ShapeDtypeStruct((1, B), jnp.float32)] * 3
    rep = pl.BlockSpec((1, B), lambda i: (0, 0))
    pf, pb, scores = pl.pallas_call(
        _fused_kernel,
        grid=(N_BLKS,),
        in_specs=[
            pl.BlockSpec((L_BLK, D, B), lambda i: (i, 0, 0)),
            pl.BlockSpec((L_BLK, 1, B), lambda i: (i, 0, 0)),
            rep,
            rep,
            pl.BlockSpec((2 * H, D), lambda i: (0, 0)),
            pl.BlockSpec((2 * H, B), lambda i: (0, 0)),
            pl.BlockSpec((2 * A, 2 * H), lambda i: (0, 0)),
            pl.BlockSpec((2 * A, B), lambda i: (0, 0)),
        ],
        out_specs=[rep, rep, rep],
        out_shape=out_shape,
        scratch_shapes=[
            pltpu.VMEM((1, B), jnp.float32),
            pltpu.VMEM((1, B), jnp.float32),
            pltpu.VMEM((1, B), jnp.float32),
            pltpu.VMEM((1, B), jnp.float32),
        ],
        compiler_params=pltpu.CompilerParams(
            dimension_semantics=("arbitrary",),
        ),
    )(states_t, actions3, lengths2, logr2, w1, b1, w2, b2)
    return pf[0], pb[0], scores[0]


# speedup vs baseline: 3.1233x; 1.0916x over previous
import jax
import jax.numpy as jnp
from jax.experimental import pallas as pl
from jax.experimental.pallas import tpu as pltpu

L, B, D, H, A = 512, 1024, 64, 64, 32
FILL = 0.0
LOG_REWARD_CLIP_MIN = -100.0

L_BLK = 32
N_BLKS = L // L_BLK


def _fused_kernel(states_ref, actions_ref, lengths_ref, logr_ref,
                  w1_ref, b1_ref, w2_ref, b2_ref,
                  pf_out, pb_out, scores_out,
                  gf_ref, gb_ref):
    i = pl.program_id(0)
    lengths = lengths_ref[...]
    w1 = w1_ref[...]
    w2 = w2_ref[...]
    b1 = b1_ref[...]
    b2 = b2_ref[...]

    acc_gf = jnp.zeros((1, B), jnp.float32)
    acc_gb = jnp.zeros((1, B), jnp.float32)

    for j in range(L_BLK):
        xt = states_ref[j]
        ht = jnp.maximum(
            jnp.dot(w1, xt,
                    preferred_element_type=jnp.float32
                    ).astype(jnp.bfloat16) + b1,
            jnp.bfloat16(0))
        lt = (jnp.dot(w2, ht, preferred_element_type=jnp.float32)
              + b2)

        e = jnp.exp(lt)
        s_f = jnp.sum(e[:A, :], axis=0, keepdims=True)
        s_b = jnp.sum(e[A:, :], axis=0, keepdims=True)
        lse_f = jnp.log(s_f)
        lse_b = jnp.log(s_b)

        a = actions_ref[j]
        step = i * L_BLK + j
        valid = step < lengths
        validb = valid & (step != lengths - 1)
        a_lo = jnp.bitwise_and(a, 7)
        a_hi = jnp.right_shift(a, 3)

        def gather_half(base):
            parts = [
                jnp.take_along_axis(lt[base + 8 * k: base + 8 * (k + 1), :],
                                    a_lo, axis=0)
                for k in range(A // 8)
            ]
            g = parts[0]
            for k in range(1, A // 8):
                g = jnp.where(a_hi == k, parts[k], g)
            return g

        g_f = gather_half(0)
        g_b = gather_half(A)

        acc_gf = acc_gf + jnp.where(valid, g_f - lse_f, FILL)
        acc_gb = acc_gb + jnp.where(validb, g_b - lse_b, FILL)

    @pl.when(i == 0)
    def _init():
        gf_ref[...] = acc_gf
        gb_ref[...] = acc_gb

    @pl.when(i > 0)
    def _acc():
        gf_ref[...] += acc_gf
        gb_ref[...] += acc_gb

    @pl.when(i == N_BLKS - 1)
    def _final():
        pf = gf_ref[...]
        pb = gb_ref[...]
        log_r = jnp.maximum(logr_ref[...], LOG_REWARD_CLIP_MIN)
        pf_out[...] = pf
        pb_out[...] = pb
        scores_out[...] = pf - pb - log_r


@jax.jit
def kernel(states, log_rewards, pf_W1, pf_b1, pf_W2, pf_b2,
           pb_W1, pb_b1, pb_W2, pb_b2, actions, lengths):
    states_t = jnp.transpose(states, (0, 2, 1))
    w1 = jnp.concatenate([pf_W1, pb_W1], axis=1).T
    b1 = jnp.broadcast_to(
        jnp.concatenate([pf_b1, pb_b1])[:, None],
        (2 * H, B)).astype(jnp.bfloat16)
    w2 = jnp.zeros((2 * H, 2 * A), jnp.float32)
    w2 = (w2.at[:H, :A].set(pf_W2).at[H:, A:].set(pb_W2).T
          .astype(jnp.bfloat16))
    b2 = jnp.broadcast_to(
        jnp.concatenate([pf_b2, pb_b2])[:, None], (2 * A, B))
    actions3 = actions[:, None, :]
    lengths2 = lengths[None, :]
    logr2 = log_rewards[None, :]

    out_shape = [jax.ShapeDtypeStruct((1, B), jnp.float32)] * 3
    rep = pl.BlockSpec((1, B), lambda i: (0, 0))
    pf, pb, scores = pl.pallas_call(
        _fused_kernel,
        grid=(N_BLKS,),
        in_specs=[
            pl.BlockSpec((L_BLK, D, B), lambda i: (i, 0, 0)),
            pl.BlockSpec((L_BLK, 1, B), lambda i: (i, 0, 0)),
            rep,
            rep,
            pl.BlockSpec((2 * H, D), lambda i: (0, 0)),
            pl.BlockSpec((2 * H, B), lambda i: (0, 0)),
            pl.BlockSpec((2 * A, 2 * H), lambda i: (0, 0)),
            pl.BlockSpec((2 * A, B), lambda i: (0, 0)),
        ],
        out_specs=[rep, rep, rep],
        out_shape=out_shape,
        scratch_shapes=[
            pltpu.VMEM((1, B), jnp.float32),
            pltpu.VMEM((1, B), jnp.float32),
        ],
        compiler_params=pltpu.CompilerParams(
            dimension_semantics=("arbitrary",),
        ),
    )(states_t, actions3, lengths2, logr2, w1, b1, w2, b2)
    return pf[0], pb[0], scores[0]
